# baseline (device time: 44493 ns/iter reference)
import jax
import jax.numpy as jnp
from jax import lax
from jax.experimental import pallas as pl
from jax.experimental.pallas import tpu as pltpu

N_DEV = 4
N_CHUNKS = 4


def kernel(t):
    m, n = t.shape
    mh = m // 2
    mq = m // 4
    me = m // 8
    nc = n // N_CHUNKS

    def body(x_ref, out_ref, s1a, s1b, r1a, r1b, s2a, s2b, r2a, r2b,
             ssem, rsem):
        my = lax.axis_index("i")
        p1 = my ^ 1
        p2 = 3 - my
        b1 = (my ^ (my // 2)) % 2
        b2 = (my // 2) % 2

        f32 = jnp.float32
        bf16 = jnp.bfloat16
        qa_start = b1 * mq + b2 * me
        qb_start = mh + b2 * mq + b1 * me

        barrier_sem = pltpu.get_barrier_semaphore()
        for nbr in [p1, p2]:
            pl.semaphore_signal(
                barrier_sem, inc=1,
                device_id=(nbr,), device_id_type=pl.DeviceIdType.MESH,
            )
        pl.semaphore_wait(barrier_sem, 2)

        def exch(idx, src, dst, partner):
            rdma = pltpu.make_async_remote_copy(
                src_ref=src, dst_ref=dst,
                send_sem=ssem.at[idx], recv_sem=rsem.at[idx],
                device_id=(partner,), device_id_type=pl.DeviceIdType.MESH,
            )
            rdma.start()
            return rdma

        def cs(c):
            return pl.ds(c * nc, nc)


        def rs1_issue(c):
            s1a[:, cs(c)] = x_ref[pl.ds((1 - b1) * mq, mq), cs(c)
                                  ].astype(bf16)
            da = exch(0 * N_CHUNKS + c, s1a.at[:, cs(c)],
                      r1a.at[:, cs(c)], p1)
            s1b[:, cs(c)] = x_ref[pl.ds(mh + (1 - b2) * mq, mq), cs(c)
                                  ].astype(bf16)
            db = exch(1 * N_CHUNKS + c, s1b.at[:, cs(c)],
                      r1b.at[:, cs(c)], p2)
            return da, db

        def rs2_issue(c):
            s2a[:, cs(c)] = (
                x_ref[pl.ds(b1 * mq + (1 - b2) * me, me), cs(c)]
                + r1a[pl.ds((1 - b2) * me, me), cs(c)].astype(f32)
            ).astype(bf16)
            da = exch(2 * N_CHUNKS + c, s2a.at[:, cs(c)],
                      r2a.at[:, cs(c)], p2)
            s2b[:, cs(c)] = (
                x_ref[pl.ds(mh + b2 * mq + (1 - b1) * me, me), cs(c)]
                + r1b[pl.ds((1 - b1) * me, me), cs(c)].astype(f32)
            ).astype(bf16)
            db = exch(3 * N_CHUNKS + c, s2b.at[:, cs(c)],
                      r2b.at[:, cs(c)], p1)
            return da, db

        def f(s):
            r = jnp.maximum(s, 0.0)
            return jnp.tanh(s) * s * s + r * r * r

        def ag1_issue(c):
            out_ref[pl.ds(qa_start, me), cs(c)] = f(
                x_ref[pl.ds(b1 * mq + b2 * me, me), cs(c)]
                + r1a[pl.ds(b2 * me, me), cs(c)].astype(f32)
                + r2a[:, cs(c)].astype(f32)
            ).astype(bf16)
            da = exch(4 * N_CHUNKS + c,
                      out_ref.at[pl.ds(qa_start, me), cs(c)],
                      out_ref.at[pl.ds(qa_start, me), cs(c)], p2)
            out_ref[pl.ds(qb_start, me), cs(c)] = f(
                x_ref[pl.ds(mh + b2 * mq + b1 * me, me), cs(c)]
                + r1b[pl.ds(b1 * me, me), cs(c)].astype(f32)
                + r2b[:, cs(c)].astype(f32)
            ).astype(bf16)
            db = exch(5 * N_CHUNKS + c,
                      out_ref.at[pl.ds(qb_start, me), cs(c)],
                      out_ref.at[pl.ds(qb_start, me), cs(c)], p1)
            return da, db

        def ag2_issue(c):
            da = exch(6 * N_CHUNKS + c,
                      out_ref.at[pl.ds(b1 * mq, mq), cs(c)],
                      out_ref.at[pl.ds(b1 * mq, mq), cs(c)], p1)
            db = exch(7 * N_CHUNKS + c,
                      out_ref.at[pl.ds(mh + b2 * mq, mq), cs(c)],
                      out_ref.at[pl.ds(mh + b2 * mq, mq), cs(c)], p2)
            return da, db

        def wait(pair):
            pair[0].wait()
            pair[1].wait()

        rs1 = [rs1_issue(c) for c in range(N_CHUNKS)]
        rs2 = []
        for c in range(N_CHUNKS):
            wait(rs1[c])
            rs2.append(rs2_issue(c))
        ag1 = []
        for c in range(N_CHUNKS):
            wait(rs2[c])
            ag1.append(ag1_issue(c))
        ag2 = []
        for c in range(N_CHUNKS):
            wait(ag1[c])
            ag2.append(ag2_issue(c))
        for c in range(N_CHUNKS):
            wait(ag2[c])

    return pl.pallas_call(
        body,
        out_shape=jax.ShapeDtypeStruct((m, n), jnp.bfloat16),
        in_specs=[pl.BlockSpec(memory_space=pltpu.VMEM)],
        out_specs=pl.BlockSpec(memory_space=pltpu.VMEM),
        scratch_shapes=[
            pltpu.VMEM((mq, n), jnp.bfloat16),
            pltpu.VMEM((mq, n), jnp.bfloat16),
            pltpu.VMEM((mq, n), jnp.bfloat16),
            pltpu.VMEM((mq, n), jnp.bfloat16),
            pltpu.VMEM((me, n), jnp.bfloat16),
            pltpu.VMEM((me, n), jnp.bfloat16),
            pltpu.VMEM((me, n), jnp.bfloat16),
            pltpu.VMEM((me, n), jnp.bfloat16),
            pltpu.SemaphoreType.DMA((8 * N_CHUNKS,)),
            pltpu.SemaphoreType.DMA((8 * N_CHUNKS,)),
        ],
        compiler_params=pltpu.CompilerParams(collective_id=0),
    )(t)


# device time: 43101 ns/iter; 1.0323x vs baseline; 1.0323x over previous
import jax
import jax.numpy as jnp
from jax import lax
from jax.experimental import pallas as pl
from jax.experimental.pallas import tpu as pltpu

N_DEV = 4
N_CHUNKS = 2


def kernel(t):
    m, n = t.shape
    mh = m // 2
    mq = m // 4
    me = m // 8
    nc = n // N_CHUNKS

    def body(x_ref, out_ref, s1a, s1b, r1a, r1b, s2a, s2b, r2a, r2b,
             ssem, rsem):
        my = lax.axis_index("i")
        p1 = my ^ 1
        p2 = 3 - my
        b1 = (my ^ (my // 2)) % 2
        b2 = (my // 2) % 2

        f32 = jnp.float32
        bf16 = jnp.bfloat16
        qa_start = b1 * mq + b2 * me
        qb_start = mh + b2 * mq + b1 * me

        barrier_sem = pltpu.get_barrier_semaphore()
        for nbr in [p1, p2]:
            pl.semaphore_signal(
                barrier_sem, inc=1,
                device_id=(nbr,), device_id_type=pl.DeviceIdType.MESH,
            )
        pl.semaphore_wait(barrier_sem, 2)

        def exch(idx, src, dst, partner):
            rdma = pltpu.make_async_remote_copy(
                src_ref=src, dst_ref=dst,
                send_sem=ssem.at[idx], recv_sem=rsem.at[idx],
                device_id=(partner,), device_id_type=pl.DeviceIdType.MESH,
            )
            rdma.start()
            return rdma

        def cs(c):
            return pl.ds(c * nc, nc)


        def rs1_issue(c):
            s1a[:, cs(c)] = x_ref[pl.ds((1 - b1) * mq, mq), cs(c)
                                  ].astype(bf16)
            da = exch(0 * N_CHUNKS + c, s1a.at[:, cs(c)],
                      r1a.at[:, cs(c)], p1)
            s1b[:, cs(c)] = x_ref[pl.ds(mh + (1 - b2) * mq, mq), cs(c)
                                  ].astype(bf16)
            db = exch(1 * N_CHUNKS + c, s1b.at[:, cs(c)],
                      r1b.at[:, cs(c)], p2)
            return da, db

        def rs2_issue(c):
            s2a[:, cs(c)] = (
                x_ref[pl.ds(b1 * mq + (1 - b2) * me, me), cs(c)]
                + r1a[pl.ds((1 - b2) * me, me), cs(c)].astype(f32)
            ).astype(bf16)
            da = exch(2 * N_CHUNKS + c, s2a.at[:, cs(c)],
                      r2a.at[:, cs(c)], p2)
            s2b[:, cs(c)] = (
                x_ref[pl.ds(mh + b2 * mq + (1 - b1) * me, me), cs(c)]
                + r1b[pl.ds((1 - b1) * me, me), cs(c)].astype(f32)
            ).astype(bf16)
            db = exch(3 * N_CHUNKS + c, s2b.at[:, cs(c)],
                      r2b.at[:, cs(c)], p1)
            return da, db

        def f(s):
            r = jnp.maximum(s, 0.0)
            return jnp.tanh(s) * s * s + r * r * r

        def ag1_issue(c):
            out_ref[pl.ds(qa_start, me), cs(c)] = f(
                x_ref[pl.ds(b1 * mq + b2 * me, me), cs(c)]
                + r1a[pl.ds(b2 * me, me), cs(c)].astype(f32)
                + r2a[:, cs(c)].astype(f32)
            ).astype(bf16)
            da = exch(4 * N_CHUNKS + c,
                      out_ref.at[pl.ds(qa_start, me), cs(c)],
                      out_ref.at[pl.ds(qa_start, me), cs(c)], p2)
            out_ref[pl.ds(qb_start, me), cs(c)] = f(
                x_ref[pl.ds(mh + b2 * mq + b1 * me, me), cs(c)]
                + r1b[pl.ds(b1 * me, me), cs(c)].astype(f32)
                + r2b[:, cs(c)].astype(f32)
            ).astype(bf16)
            db = exch(5 * N_CHUNKS + c,
                      out_ref.at[pl.ds(qb_start, me), cs(c)],
                      out_ref.at[pl.ds(qb_start, me), cs(c)], p1)
            return da, db

        def ag2_issue(c):
            da = exch(6 * N_CHUNKS + c,
                      out_ref.at[pl.ds(b1 * mq, mq), cs(c)],
                      out_ref.at[pl.ds(b1 * mq, mq), cs(c)], p1)
            db = exch(7 * N_CHUNKS + c,
                      out_ref.at[pl.ds(mh + b2 * mq, mq), cs(c)],
                      out_ref.at[pl.ds(mh + b2 * mq, mq), cs(c)], p2)
            return da, db

        def wait(pair):
            pair[0].wait()
            pair[1].wait()

        rs1 = [rs1_issue(c) for c in range(N_CHUNKS)]
        rs2 = []
        for c in range(N_CHUNKS):
            wait(rs1[c])
            rs2.append(rs2_issue(c))
        ag1 = []
        for c in range(N_CHUNKS):
            wait(rs2[c])
            ag1.append(ag1_issue(c))
        ag2 = []
        for c in range(N_CHUNKS):
            wait(ag1[c])
            ag2.append(ag2_issue(c))
        for c in range(N_CHUNKS):
            wait(ag2[c])

    return pl.pallas_call(
        body,
        out_shape=jax.ShapeDtypeStruct((m, n), jnp.bfloat16),
        in_specs=[pl.BlockSpec(memory_space=pltpu.VMEM)],
        out_specs=pl.BlockSpec(memory_space=pltpu.VMEM),
        scratch_shapes=[
            pltpu.VMEM((mq, n), jnp.bfloat16),
            pltpu.VMEM((mq, n), jnp.bfloat16),
            pltpu.VMEM((mq, n), jnp.bfloat16),
            pltpu.VMEM((mq, n), jnp.bfloat16),
            pltpu.VMEM((me, n), jnp.bfloat16),
            pltpu.VMEM((me, n), jnp.bfloat16),
            pltpu.VMEM((me, n), jnp.bfloat16),
            pltpu.VMEM((me, n), jnp.bfloat16),
            pltpu.SemaphoreType.DMA((8 * N_CHUNKS,)),
            pltpu.SemaphoreType.DMA((8 * N_CHUNKS,)),
        ],
        compiler_params=pltpu.CompilerParams(collective_id=0),
    )(t)
